# baseline (speedup 1.0000x reference)
import functools

import jax
import jax.numpy as jnp
from jax import lax
from jax.experimental import pallas as pl
from jax.experimental.pallas import tpu as pltpu
from jax.experimental.pallas import tpu_sc as plsc

_BATCH = 16384
_EMBED = 64
_PACK = 128
_NUM_CORES = 2
_NUM_SUBCORES = 16
_NW = _NUM_CORES * _NUM_SUBCORES
_BPW = _BATCH // _NW
_CHUNK = 128
_NCHUNK = _BPW // _CHUNK

_mesh = plsc.VectorSubcoreMesh(core_axis_name="c", subcore_axis_name="s")


@functools.partial(
    pl.kernel,
    mesh=_mesh,
    compiler_params=pltpu.CompilerParams(needs_layout_passes=False),
    out_type=jax.ShapeDtypeStruct((_BATCH,), jnp.float32),
    scratch_types=[
        pltpu.VMEM((_NCHUNK, _CHUNK), jnp.int32),
        pltpu.VMEM((_NCHUNK, _CHUNK), jnp.int32),
        pltpu.VMEM((_NCHUNK, _CHUNK), jnp.int32),
        pltpu.VMEM((_NCHUNK, _CHUNK), jnp.int32),
        pltpu.VMEM((2, _CHUNK, _PACK), jnp.float32),
        pltpu.VMEM((2, _CHUNK, _PACK), jnp.float32),
        pltpu.VMEM((_BPW,), jnp.float32),
        pltpu.SemaphoreType.DMA,
        pltpu.SemaphoreType.DMA,
    ],
)
def _sc_kernel(uid_hbm, iid_hbm, ut_hbm, it_hbm, out_hbm,
               uid_v, iid_v, urow_v, irow_v, ubuf, ibuf, out_v,
               sem0, sem1):
    wid = lax.axis_index("s") * _NUM_CORES + lax.axis_index("c")
    base = wid * _BPW

    for j in range(_NCHUNK):
        pltpu.sync_copy(uid_hbm.at[pl.ds(base + j * _CHUNK, _CHUNK)],
                        uid_v.at[j])
        pltpu.sync_copy(iid_hbm.at[pl.ds(base + j * _CHUNK, _CHUNK)],
                        iid_v.at[j])

    for j in range(_NCHUNK):
        for s in range(_CHUNK // 16):
            sl = pl.ds(s * 16, 16)
            urow_v[j, sl] = jax.lax.shift_right_logical(uid_v[j, sl], 1)
            irow_v[j, sl] = jax.lax.shift_right_logical(iid_v[j, sl], 1)

    sems = (sem0, sem1)

    def fire(j):
        cu = pltpu.async_copy(ut_hbm.at[urow_v.at[j]], ubuf.at[j % 2],
                              sems[j % 2])
        ci = pltpu.async_copy(it_hbm.at[irow_v.at[j]], ibuf.at[j % 2],
                              sems[j % 2])
        return (cu, ci)

    lanes = lax.iota(jnp.int32, 16)
    inflight = [fire(0), fire(1)]

    for j in range(_NCHUNK):
        cu, ci = inflight[j]
        cu.wait()
        ci.wait()

        ub = ubuf.at[j % 2]
        ib = ibuf.at[j % 2]

        def group_body(g, carry, j=j, ub=ub, ib=ib):
            sl = pl.ds(g * 16, 16)
            row_idx = g * 16 + lanes
            ucol = jax.lax.bitwise_and(uid_v[j, sl], 1) * _EMBED
            icol = jax.lax.bitwise_and(iid_v[j, sl], 1) * _EMBED
            acc = jnp.zeros((16,), jnp.float32)
            for c in range(_EMBED):
                u = plsc.load_gather(ub, [row_idx, ucol + c])
                v = plsc.load_gather(ib, [row_idx, icol + c])
                acc = acc + u * v
            out_v[pl.ds(j * _CHUNK + g * 16, 16)] = acc
            return carry

        lax.fori_loop(0, _CHUNK // 16, group_body, 0)

        if j + 2 < _NCHUNK:
            inflight.append(fire(j + 2))

    pltpu.sync_copy(out_v, out_hbm.at[pl.ds(base, _BPW)])


def kernel(user_ids, item_ids, user_table, item_table):
    ut2 = user_table.reshape(500000, _PACK)
    it2 = item_table.reshape(500000, _PACK)
    return _sc_kernel(user_ids, item_ids, ut2, it2)
